# baseline (speedup 1.0000x reference)
import functools

import jax
import jax.numpy as jnp
import numpy as np
from jax import lax
from jax.experimental import pallas as pl
from jax.experimental.pallas import tpu as pltpu
from jax.experimental.pallas import tpu_sc as plsc

E = 16
NPROJ = 2
B = 32
PE = 1024
EXPERT_DIM = 128
DEPTH = 8
LW = 3
L = 16


def _cantor_coord(position, max_len, depth):
    x = position / max(1, max_len - 1)
    x = max(1e-06, min(x, 1.0 - 1e-06))
    val = 0.0
    factor = 0.5
    for _ in range(depth):
        x *= 3.0
        digit = int(x)
        x -= digit
        if digit == 2:
            val += factor
        factor *= 0.5
    return val


def _routes():
    coords = np.array([_cantor_coord(i, E, DEPTH) for i in range(E)], dtype=np.float32)
    routes = np.zeros((E, LW), dtype=np.int32)
    for i in range(E):
        d = np.abs(coords - coords[i])
        routes[i] = np.argsort(d, kind='stable')[:LW]
    return routes


_ROUTES = _routes()
_IS_SELF = (_ROUTES == np.arange(E)[:, None]).astype(np.float32)
_SCALE = 0.25 / np.sqrt(float(EXPERT_DIM))


def _sc_body(qr, kr, vr, bsel, isself, tvec, out, qb, kb, vb, ob, sv, sem0, sem1):
    b = lax.axis_index("s") * 2 + lax.axis_index("c")

    pltpu.sync_copy(bsel, sv.at[0])
    pltpu.sync_copy(isself, sv.at[1])
    pltpu.sync_copy(tvec, sv.at[2, 0])
    t = sv[2, 0, :]
    scale = _SCALE / jnp.abs(t)
    fvecs = []
    for n in range(LW):
        bs = sv[0, n, :]
        sig = 1.0 / (1.0 + jnp.exp(-bs))
        iss = sv[1, n, :]
        fvecs.append((iss + (1.0 - iss) * sig) * scale)

    sems = (sem0, sem1)

    def fire(e, s):
        handles = []
        sem = sems[s]
        for p in range(NPROJ):
            cp = pltpu.make_async_copy(qr.at[e * (NPROJ * B) + p * B + b], qb.at[s, p], sem)
            cp.start()
            handles.append(cp)
        for n in range(LW):
            r = int(_ROUTES[e, n])
            for p in range(NPROJ):
                row = r * (NPROJ * B) + p * B + b
                cp = pltpu.make_async_copy(kr.at[row], kb.at[s, n * NPROJ + p], sem)
                cp.start()
                handles.append(cp)
                cp = pltpu.make_async_copy(vr.at[row], vb.at[s, n * NPROJ + p], sem)
                cp.start()
                handles.append(cp)
        return handles

    def compute(e, s):
        f0 = fvecs[0][e]
        f1 = fvecs[1][e]
        f2 = fvecs[2][e]

        def chunk(j, carry):
            sl = pl.ds(j * L, L)
            q = qb[s, 0, sl] + qb[s, 1, sl]
            s0 = q * (kb[s, 0, sl] + kb[s, 1, sl]) * f0
            s1 = q * (kb[s, 2, sl] + kb[s, 3, sl]) * f1
            s2 = q * (kb[s, 4, sl] + kb[s, 5, sl]) * f2
            m = jnp.maximum(s0, jnp.maximum(s1, s2))
            w0 = jnp.exp(s0 - m)
            w1 = jnp.exp(s1 - m)
            w2 = jnp.exp(s2 - m)
            r = 0.5 / (w0 + w1 + w2)
            acc = w0 * (vb[s, 0, sl] + vb[s, 1, sl])
            acc = acc + w1 * (vb[s, 2, sl] + vb[s, 3, sl])
            acc = acc + w2 * (vb[s, 4, sl] + vb[s, 5, sl])
            ob[e, sl] = acc * r
            return carry

        lax.fori_loop(0, PE // L, chunk, 0)

    pending = fire(0, 0)
    for e in range(E):
        nxt = fire(e + 1, (e + 1) % 2) if e + 1 < E else None
        for cp in pending:
            cp.wait()
        compute(e, e % 2)
        pending = nxt

    pltpu.sync_copy(ob, out.at[b])


@jax.jit
def _run(proj_Q, proj_K, proj_V, beta_logits, temperature):
    qr = proj_Q.reshape(E * NPROJ * B, PE)
    kr = proj_K.reshape(E * NPROJ * B, PE)
    vr = proj_V.reshape(E * NPROJ * B, PE)
    bsel = beta_logits[np.arange(E)[:, None], _ROUTES].T
    isself = jnp.asarray(_IS_SELF.T)
    tvec = jnp.broadcast_to(jnp.reshape(jnp.abs(temperature), (1,)), (L,))

    kfun = pl.kernel(
        _sc_body,
        out_type=jax.ShapeDtypeStruct((B, E, PE), jnp.float32),
        mesh=plsc.VectorSubcoreMesh(core_axis_name="c", subcore_axis_name="s",
                                    num_cores=2, num_subcores=16),
        scratch_types=[
            pltpu.VMEM((2, NPROJ, PE), jnp.float32),
            pltpu.VMEM((2, LW * NPROJ, PE), jnp.float32),
            pltpu.VMEM((2, LW * NPROJ, PE), jnp.float32),
            pltpu.VMEM((E, PE), jnp.float32),
            pltpu.VMEM((3, LW, L), jnp.float32),
            pltpu.SemaphoreType.DMA,
            pltpu.SemaphoreType.DMA,
        ],
    )
    out = kfun(qr, kr, vr, bsel, isself, tvec)
    return out.reshape(B, E * PE)


def kernel(proj_Q, proj_K, proj_V, beta_logits, temperature, num_patches):
    del num_patches
    return _run(proj_Q, proj_K, proj_V, beta_logits, temperature)
